# Initial kernel scaffold; baseline (speedup 1.0000x reference)
import jax
import jax.numpy as jnp
from jax.experimental import pallas as pl


def kernel(x, target):
    raise NotImplementedError("write your pallas kernel here")


# speedup vs baseline: 1.7767x; 1.7767x over previous
import functools

import jax
import jax.numpy as jnp
from jax.experimental import pallas as pl
from jax.experimental.pallas import tpu as pltpu

_SIZE = 100000
_PAD = 0
_SMOOTHING = 0.1
_CONF = 1.0 - _SMOOTHING
_EPS = _SMOOTHING / (_SIZE - 2)

_B = 1024
_CB = 2048
_NCB = (_SIZE + _CB - 1) // _CB


def _loss_body(t_ref, x_ref, o_ref):
    j = pl.program_id(0)
    c0 = j * _CB
    x = x_ref[...]
    t = t_ref[...]
    col = jax.lax.broadcasted_iota(jnp.int32, (_B, _CB), 1) + c0
    inv_b = 1.0 / _B
    w = jnp.where(col == t, -_CONF * inv_b, -_EPS * inv_b)
    valid = (col < _SIZE) & (col != _PAD) & (t != _PAD)
    partial = jnp.sum(jnp.where(valid, w * x, 0.0))

    @pl.when(j == 0)
    def _init():
        n_nonpad = jnp.sum((t != _PAD).astype(jnp.float32))
        c1 = _SMOOTHING * jnp.log(jnp.float32(_EPS)) + _CONF * jnp.log(
            jnp.float32(_CONF))
        o_ref[0, 0] = n_nonpad * c1 * inv_b

    o_ref[0, 0] += partial


@jax.jit
def kernel(x, target):
    t2 = target.astype(jnp.int32).reshape(_B, 1)
    out = pl.pallas_call(
        _loss_body,
        grid=(_NCB,),
        in_specs=[
            pl.BlockSpec((_B, 1), lambda j: (0, 0)),
            pl.BlockSpec((_B, _CB), lambda j: (0, j)),
        ],
        out_specs=pl.BlockSpec(memory_space=pltpu.SMEM),
        out_shape=jax.ShapeDtypeStruct((1, 1), jnp.float32),
    )(t2, x)
    return out[0, 0]
